# baseline (speedup 1.0000x reference)
import jax
import jax.numpy as jnp
from jax import lax
from jax.experimental import pallas as pl
from jax.experimental.pallas import tpu as pltpu
from jax.experimental.pallas import tpu_sc as plsc
import functools

NUM_EMB = 1_000_000
DIM = 64
BATCH = 16384
FIELDS = 26
FPAD = 32
TOT = BATCH * FIELDS

NC = 2
NS = 16
NW = NC * NS
BPW = BATCH // NW
LPW = BPW * FIELDS
L = 16

_mesh = plsc.VectorSubcoreMesh(
    core_axis_name="c", subcore_axis_name="s", num_cores=NC, num_subcores=NS
)


@functools.partial(
    pl.kernel,
    out_type=jax.ShapeDtypeStruct((BATCH * FPAD, DIM), jnp.float32),
    mesh=_mesh,
    scratch_types=[
        pltpu.VMEM((LPW,), jnp.int32),
        pltpu.VMEM((FIELDS * 8, DIM), jnp.float32),
        pltpu.VMEM((FIELDS * 8, DIM), jnp.float32),
        pltpu.VMEM((FPAD, DIM), jnp.float32),
        pltpu.VMEM((FPAD, DIM), jnp.float32),
        pltpu.SemaphoreType.DMA,
        pltpu.SemaphoreType.DMA,
    ],
    compiler_params=pltpu.CompilerParams(needs_layout_passes=False),
)
def _sc_gather(tbl, idx, out, idx_v, blk0, blk1, st0, st1, sem0, sem1):
    wid = lax.axis_index("s") * NC + lax.axis_index("c")
    pltpu.sync_copy(idx.at[pl.ds(wid * LPW, LPW)], idx_v)

    blks = (blk0, blk1)
    stgs = (st0, st1)
    sems = (sem0, sem1)
    lanes = lax.iota(jnp.int32, L)

    def get_idx(j, k):
        jj = j * FIELDS + k
        base = (jj // L) * L
        vec = idx_v[pl.ds(base, L)]
        return jnp.max(jnp.where(lanes == jj - base, vec, jnp.int32(-1)))

    def issue(j, p):
        @pl.loop(0, FIELDS)
        def _(k):
            r = get_idx(j, k)
            r8 = (r // 8) * 8
            pltpu.make_async_copy(
                tbl.at[pl.ds(r8, 8), :], blks[p].at[pl.ds(k * 8, 8), :], sems[p]
            ).start()

    def drain(p):
        pltpu.make_async_copy(tbl.at[pl.ds(0, FIELDS * 8), :], blks[p], sems[p]).wait()

    def extract_and_store(j, p):
        @pl.loop(0, FIELDS)
        def _(k):
            r = get_idx(j, k)
            s = r - (r // 8) * 8
            rr = k * 8 + s
            for c in range(DIM // L):
                stgs[p][k, pl.ds(c * L, L)] = blks[p][rr, pl.ds(c * L, L)]
        b = wid * BPW + j
        pltpu.sync_copy(stgs[p], out.at[pl.ds(b * FPAD, FPAD)])

    issue(0, 0)

    @pl.loop(0, BPW // 2)
    def _(jp):
        j0 = 2 * jp
        issue(j0 + 1, 1)
        drain(0)
        extract_and_store(j0, 0)

        @pl.when(jp < BPW // 2 - 1)
        def _():
            issue(j0 + 2, 0)

        drain(1)
        extract_and_store(j0 + 1, 1)


def kernel(input, weight):
    idx = input.reshape(-1).astype(jnp.int32)
    out = _sc_gather(weight, idx)
    return out.reshape(BATCH, FPAD, DIM)[:, :FIELDS, :]
